# Initial kernel scaffold; baseline (speedup 1.0000x reference)
import jax
import jax.numpy as jnp
from jax.experimental import pallas as pl


def kernel(edge_feat, triplets, W1, b1, ln1_w, ln1_b, W2, b2, ln2_w, ln2_b, Wpsi, bpsi, norm_w, norm_b):
    raise NotImplementedError("write your pallas kernel here")


# speedup vs baseline: 2.4490x; 2.4490x over previous
import functools

import jax
import jax.numpy as jnp
from jax import lax
from jax.experimental import pallas as pl
from jax.experimental.pallas import tpu as pltpu
from jax.experimental.pallas import tpu_sc as plsc

E = 800000
T = 1600000
EDGE_DIM = 16
HIDDEN = 64


_NC, _NS, _L = 2, 16, 16
_NW = _NC * _NS


def _gather_body(table_hbm, idx_hbm, out_hbm, idx_v, rows_v, sem):
    n_rows = 2 * T
    per_w = n_rows // _NW
    blk = 2000
    n_blk = per_w // blk
    wid = lax.axis_index("s") * _NC + lax.axis_index("c")
    base = wid * per_w

    def body(i, carry):
        off = base + i * blk
        pltpu.sync_copy(idx_hbm.at[pl.ds(off, blk)], idx_v)
        pltpu.async_copy(table_hbm.at[idx_v], rows_v, sem).wait()
        pltpu.sync_copy(rows_v, out_hbm.at[pl.ds(off, blk)])
        return carry

    lax.fori_loop(0, n_blk, body, 0)


def _sc_gather(table, idx):
    mesh = plsc.VectorSubcoreMesh(core_axis_name="c", subcore_axis_name="s")
    blk = 2000
    f = pl.kernel(
        _gather_body,
        out_type=jax.ShapeDtypeStruct((2 * T, EDGE_DIM), jnp.float32),
        mesh=mesh,
        scratch_types=[
            pltpu.VMEM((blk,), jnp.int32),
            pltpu.VMEM((blk, EDGE_DIM), jnp.float32),
            pltpu.SemaphoreType.DMA,
        ],
        compiler_params=pltpu.CompilerParams(use_tc_tiling_on_sc=False),
    )
    return f(table, idx)


def _silu(x):
    return x * jax.nn.sigmoid(x)


def _ln(x, w, b, eps=1e-5):
    m = jnp.mean(x, axis=-1, keepdims=True)
    v = jnp.mean((x - m) ** 2, axis=-1, keepdims=True)
    return (x - m) * jax.lax.rsqrt(v + eps) * w + b


def _mlp_body(g1_ref, g2_ref, t4_ref, w1a_ref, w1b_ref, w1c_ref, b1_ref,
              ln1w_ref, ln1b_ref, w2_ref, b2_ref, ln2w_ref, ln2b_ref,
              wpsi_ref, o_ref):
    h = (jnp.dot(g1_ref[...], w1a_ref[...], preferred_element_type=jnp.float32)
         + jnp.dot(g2_ref[...], w1b_ref[...], preferred_element_type=jnp.float32)
         + jnp.dot(t4_ref[...], w1c_ref[...], preferred_element_type=jnp.float32)
         + b1_ref[...])
    h = _ln(_silu(h), ln1w_ref[...], ln1b_ref[...])
    h = jnp.dot(h, w2_ref[...], preferred_element_type=jnp.float32) + b2_ref[...]
    h = _ln(_silu(h), ln2w_ref[...], ln2b_ref[...])
    o_ref[...] = jnp.dot(h, wpsi_ref[...], preferred_element_type=jnp.float32)


def _tc_mlp(g1, g2, t4, W1, b1, ln1_w, ln1_b, W2, b2, ln2_w, ln2_b, Wpsi):
    BT = 4000
    grid = (T // BT,)
    w1a = W1[:EDGE_DIM]
    w1b = W1[EDGE_DIM:2 * EDGE_DIM]
    w1c = W1[2 * EDGE_DIM:]
    row = lambda i: (i, 0)
    full = lambda i: (0, 0)
    return pl.pallas_call(
        _mlp_body,
        grid=grid,
        in_specs=[
            pl.BlockSpec((BT, EDGE_DIM), row),
            pl.BlockSpec((BT, EDGE_DIM), row),
            pl.BlockSpec((BT, 4), row),
            pl.BlockSpec((EDGE_DIM, HIDDEN), full),
            pl.BlockSpec((EDGE_DIM, HIDDEN), full),
            pl.BlockSpec((4, HIDDEN), full),
            pl.BlockSpec((1, HIDDEN), full),
            pl.BlockSpec((1, HIDDEN), full),
            pl.BlockSpec((1, HIDDEN), full),
            pl.BlockSpec((HIDDEN, HIDDEN), full),
            pl.BlockSpec((1, HIDDEN), full),
            pl.BlockSpec((1, HIDDEN), full),
            pl.BlockSpec((1, HIDDEN), full),
            pl.BlockSpec((HIDDEN, EDGE_DIM), full),
        ],
        out_specs=pl.BlockSpec((BT, EDGE_DIM), row),
        out_shape=jax.ShapeDtypeStruct((T, EDGE_DIM), jnp.float32),
    )(g1, g2, t4, w1a, w1b, w1c, b1[None], ln1_w[None], ln1_b[None],
      W2, b2[None], ln2_w[None], ln2_b[None], Wpsi)


def _final_body(ef_ref, sc_ref, cnt_ref, bpsi_ref, nw_ref, nb_ref, o_ref):
    cnt = jnp.maximum(cnt_ref[...], 1.0)
    x = ef_ref[...] + sc_ref[...] / cnt + bpsi_ref[...]
    o_ref[...] = _ln(x, nw_ref[...], nb_ref[...])


def _tc_final(edge_feat, scat, cnt, bpsi, norm_w, norm_b):
    BE = 4000
    grid = (E // BE,)
    row = lambda i: (i, 0)
    full = lambda i: (0, 0)
    return pl.pallas_call(
        _final_body,
        grid=grid,
        in_specs=[
            pl.BlockSpec((BE, EDGE_DIM), row),
            pl.BlockSpec((BE, EDGE_DIM), row),
            pl.BlockSpec((BE, 1), row),
            pl.BlockSpec((1, EDGE_DIM), full),
            pl.BlockSpec((1, EDGE_DIM), full),
            pl.BlockSpec((1, EDGE_DIM), full),
        ],
        out_specs=pl.BlockSpec((BE, EDGE_DIM), row),
        out_shape=jax.ShapeDtypeStruct((E, EDGE_DIM), jnp.float32),
    )(edge_feat, scat, cnt[:, None], bpsi[None], norm_w[None], norm_b[None])


def kernel(edge_feat, triplets, W1, b1, ln1_w, ln1_b, W2, b2, ln2_w, ln2_b,
           Wpsi, bpsi, norm_w, norm_b):
    e_ij = triplets[:, 0].astype(jnp.int32)
    e_kj = triplets[:, 1].astype(jnp.int32)
    t4 = triplets[:, 2:6]

    idx = jnp.concatenate([e_ij, e_kj], axis=0)
    rows = _sc_gather(edge_feat, idx)
    g1 = rows[:T]
    g2 = rows[T:]

    hw = _tc_mlp(g1, g2, t4, W1, b1, ln1_w, ln1_b, W2, b2, ln2_w, ln2_b, Wpsi)

    scat = jax.ops.segment_sum(hw, e_ij, num_segments=E)
    cnt = jax.ops.segment_sum(jnp.ones((T,), jnp.float32), e_ij, num_segments=E)

    return _tc_final(edge_feat, scat, cnt, bpsi, norm_w, norm_b)


# speedup vs baseline: 3.5296x; 1.4412x over previous
import functools

import jax
import jax.numpy as jnp
from jax import lax
from jax.experimental import pallas as pl
from jax.experimental.pallas import tpu as pltpu
from jax.experimental.pallas import tpu_sc as plsc

E = 800000
T = 1600000
EDGE_DIM = 16
HIDDEN = 64


_NC, _NS, _L = 2, 16, 16
_NW = _NC * _NS


def _gather_body(table_hbm, idx_hbm, out_hbm, idx_v, rows_v, sem):
    n_rows = 2 * T
    per_w = n_rows // _NW
    blk = 2000
    n_blk = per_w // blk
    wid = lax.axis_index("s") * _NC + lax.axis_index("c")
    base = wid * per_w

    def body(i, carry):
        off = base + i * blk
        pltpu.sync_copy(idx_hbm.at[pl.ds(off, blk)], idx_v)
        pltpu.async_copy(table_hbm.at[idx_v], rows_v, sem).wait()
        pltpu.sync_copy(rows_v, out_hbm.at[pl.ds(off, blk)])
        return carry

    lax.fori_loop(0, n_blk, body, 0)


def _sc_gather(table, idx):
    mesh = plsc.VectorSubcoreMesh(core_axis_name="c", subcore_axis_name="s")
    blk = 2000
    f = pl.kernel(
        _gather_body,
        out_type=jax.ShapeDtypeStruct((2 * T, EDGE_DIM), jnp.float32),
        mesh=mesh,
        scratch_types=[
            pltpu.VMEM((blk,), jnp.int32),
            pltpu.VMEM((blk, EDGE_DIM), jnp.float32),
            pltpu.SemaphoreType.DMA,
        ],
        compiler_params=pltpu.CompilerParams(use_tc_tiling_on_sc=False),
    )
    return f(table, idx)


def _count_body(eij_hbm, out_hbm, idx_v, ones_v, zeros_v, cnt_sh, sem):
    del sem
    c = lax.axis_index("c")
    s = lax.axis_index("s")
    w = s * _NC + c

    def fill1(i, _):
        ones_v[pl.ds(i * 16, 16)] = jnp.full((16,), 1.0, jnp.float32)
        return 0

    def fill0(i, _):
        zeros_v[pl.ds(i * 16, 16)] = jnp.zeros((16,), jnp.float32)
        return 0

    lax.fori_loop(0, 125, fill1, 0)
    lax.fori_loop(0, 625, fill0, 0)
    for z in range(5):
        pltpu.sync_copy(zeros_v, cnt_sh.at[pl.ds(s * 50000 + z * 10000, 10000)])
    plsc.subcore_barrier()
    per_w = T // _NW
    for chunk in range(per_w // 2000):
        pltpu.sync_copy(eij_hbm.at[pl.ds(w * per_w + chunk * 2000, 2000)], idx_v)
        pltpu.sync_copy(ones_v, cnt_sh.at[idx_v], add=True)
    plsc.subcore_barrier()
    pltpu.sync_copy(cnt_sh.at[pl.ds(s * 50000, 50000)],
                    out_hbm.at[c, pl.ds(s * 50000, 50000)])


def _sc_count(eij):
    mesh = plsc.VectorSubcoreMesh(core_axis_name="c", subcore_axis_name="s")
    f = pl.kernel(
        _count_body,
        out_type=jax.ShapeDtypeStruct((2, E), jnp.float32),
        mesh=mesh,
        scratch_types=[
            pltpu.VMEM((2000,), jnp.int32),
            pltpu.VMEM((2000,), jnp.float32),
            pltpu.VMEM((10000,), jnp.float32),
            pltpu.VMEM_SHARED((E,), jnp.float32),
            pltpu.SemaphoreType.DMA,
        ],
        compiler_params=pltpu.CompilerParams(use_tc_tiling_on_sc=False),
    )
    return f(eij)


_CB = 100000
_NPASS = 4
_ACC = 102400
_DUMMY = 100000
_GRP = 5
_LF = 64
_LCAP = _LF + _GRP
_CAPB = 16 * _LCAP
_SIB = 2000


def _scatter_body(eij_hbm, hw_hbm, out_hbm, accum, idx_v, offs_v, ids_v,
                  rows_v, zeros_v, sem):
    c = lax.axis_index("c")
    s = lax.axis_index("s")
    iota = lax.iota(jnp.int32, 16)

    def zfill(i, _):
        zeros_v[i] = jnp.zeros((16,), jnp.float32)
        return 0

    lax.fori_loop(0, 400, zfill, 0)

    def _refill_one(i, _):
        offs_v[pl.ds(i * 16, 16)] = jnp.full((16,), _DUMMY, jnp.int32)
        ids_v[pl.ds(i * 16, 16)] = jnp.zeros((16,), jnp.int32)
        return 0

    def refill():
        lax.fori_loop(0, _CAPB // 16, _refill_one, 0)

    lane_base = iota * _LCAP

    def flush():
        pltpu.async_copy(hw_hbm.at[ids_v], rows_v, sem).wait()
        pltpu.sync_copy(rows_v, accum.at[offs_v], add=True)
        refill()

    for p in range(_NPASS):
        lo = (c * _NPASS + p) * _CB
        for z in range(16):
            pltpu.sync_copy(zeros_v, accum.at[pl.ds(s * 6400 + z * 400, 400)])
        refill()
        plsc.subcore_barrier()

        zero16 = jnp.zeros((16,), jnp.int32)
        one16 = jnp.full((16,), 1, jnp.int32)
        cb_v = jnp.full((16,), _CB, jnp.int32)
        lf_v = jnp.full((16,), _LF, jnp.int32)
        dummy_v = jnp.full((16,), _DUMMY, jnp.int32)
        lo_v = jnp.full((16,), lo, jnp.int32)

        def chunk_body(chunk, cnt_v):
            pltpu.sync_copy(
                eij_hbm.at[pl.ds(s * (T // _NS) + chunk * _SIB, _SIB)], idx_v)

            def group_body(g, cnt_v):
                for u in range(_GRP):
                    iv = idx_v[pl.ds((g * _GRP + u) * 16, 16)]
                    off = iv - lo_v
                    m = (off >= zero16) & (off < cb_v)
                    tid0 = s * (T // _NS) + chunk * _SIB + (g * _GRP + u) * 16
                    tids = jnp.full((16,), tid0, jnp.int32) + iota
                    pos = lane_base + cnt_v
                    plsc.store_scatter(offs_v, [pos], jnp.where(m, off, dummy_v))
                    plsc.store_scatter(ids_v, [pos], jnp.where(m, tids, zero16))
                    cnt_v = cnt_v + jnp.where(m, one16, zero16)

                anyfull = plsc.all_reduce_population_count(cnt_v >= lf_v)[0]

                def do_flush(c):
                    flush()
                    return zero16

                return lax.cond(anyfull > 0, do_flush, lambda c: c, cnt_v)

            return lax.fori_loop(0, _SIB // (16 * _GRP), group_body, cnt_v)

        cnt_v = lax.fori_loop(0, (T // _NS) // _SIB, chunk_body, zero16)
        del cnt_v
        flush()
        plsc.subcore_barrier()

        @pl.when(s < 15)
        def _():
            pltpu.sync_copy(accum.at[pl.ds(s * 6256, 6256)],
                            out_hbm.at[pl.ds(lo + s * 6256, 6256)])

        @pl.when(s == 15)
        def _():
            pltpu.sync_copy(accum.at[pl.ds(15 * 6256, _CB - 15 * 6256)],
                            out_hbm.at[pl.ds(lo + 15 * 6256, _CB - 15 * 6256)])

        plsc.subcore_barrier()


def _sc_scatter(eij, hw):
    mesh = plsc.VectorSubcoreMesh(core_axis_name="c", subcore_axis_name="s")
    f = pl.kernel(
        _scatter_body,
        out_type=jax.ShapeDtypeStruct((E, EDGE_DIM), jnp.float32),
        mesh=mesh,
        scratch_types=[
            pltpu.VMEM_SHARED((_ACC, EDGE_DIM), jnp.float32),
            pltpu.VMEM((_SIB,), jnp.int32),
            pltpu.VMEM((_CAPB,), jnp.int32),
            pltpu.VMEM((_CAPB,), jnp.int32),
            pltpu.VMEM((_CAPB, EDGE_DIM), jnp.float32),
            pltpu.VMEM((400, EDGE_DIM), jnp.float32),
            pltpu.SemaphoreType.DMA,
        ],
        compiler_params=pltpu.CompilerParams(use_tc_tiling_on_sc=False,
                                             needs_layout_passes=False),
    )
    return f(eij, hw)


def _silu(x):
    return x * jax.nn.sigmoid(x)


def _ln(x, w, b, eps=1e-5):
    m = jnp.mean(x, axis=-1, keepdims=True)
    v = jnp.mean((x - m) ** 2, axis=-1, keepdims=True)
    return (x - m) * jax.lax.rsqrt(v + eps) * w + b


def _mlp_body(g1_ref, g2_ref, t4_ref, w1a_ref, w1b_ref, w1c_ref, b1_ref,
              ln1w_ref, ln1b_ref, w2_ref, b2_ref, ln2w_ref, ln2b_ref,
              wpsi_ref, o_ref):
    h = (jnp.dot(g1_ref[...], w1a_ref[...], preferred_element_type=jnp.float32)
         + jnp.dot(g2_ref[...], w1b_ref[...], preferred_element_type=jnp.float32)
         + jnp.dot(t4_ref[...], w1c_ref[...], preferred_element_type=jnp.float32)
         + b1_ref[...])
    h = _ln(_silu(h), ln1w_ref[...], ln1b_ref[...])
    h = jnp.dot(h, w2_ref[...], preferred_element_type=jnp.float32) + b2_ref[...]
    h = _ln(_silu(h), ln2w_ref[...], ln2b_ref[...])
    o_ref[...] = jnp.dot(h, wpsi_ref[...], preferred_element_type=jnp.float32)


def _tc_mlp(g1, g2, t4, W1, b1, ln1_w, ln1_b, W2, b2, ln2_w, ln2_b, Wpsi):
    BT = 4000
    grid = (T // BT,)
    w1a = W1[:EDGE_DIM]
    w1b = W1[EDGE_DIM:2 * EDGE_DIM]
    w1c = W1[2 * EDGE_DIM:]
    row = lambda i: (i, 0)
    full = lambda i: (0, 0)
    return pl.pallas_call(
        _mlp_body,
        grid=grid,
        in_specs=[
            pl.BlockSpec((BT, EDGE_DIM), row),
            pl.BlockSpec((BT, EDGE_DIM), row),
            pl.BlockSpec((BT, 4), row),
            pl.BlockSpec((EDGE_DIM, HIDDEN), full),
            pl.BlockSpec((EDGE_DIM, HIDDEN), full),
            pl.BlockSpec((4, HIDDEN), full),
            pl.BlockSpec((1, HIDDEN), full),
            pl.BlockSpec((1, HIDDEN), full),
            pl.BlockSpec((1, HIDDEN), full),
            pl.BlockSpec((HIDDEN, HIDDEN), full),
            pl.BlockSpec((1, HIDDEN), full),
            pl.BlockSpec((1, HIDDEN), full),
            pl.BlockSpec((1, HIDDEN), full),
            pl.BlockSpec((HIDDEN, EDGE_DIM), full),
        ],
        out_specs=pl.BlockSpec((BT, EDGE_DIM), row),
        out_shape=jax.ShapeDtypeStruct((T, EDGE_DIM), jnp.float32),
    )(g1, g2, t4, w1a, w1b, w1c, b1[None], ln1_w[None], ln1_b[None],
      W2, b2[None], ln2_w[None], ln2_b[None], Wpsi)


def _final_body(ef_ref, sc_ref, c0_ref, c1_ref, bpsi_ref, nw_ref, nb_ref, o_ref):
    cnt = jnp.maximum(c0_ref[...] + c1_ref[...], 1.0)
    x = ef_ref[...] + sc_ref[...] / cnt + bpsi_ref[...]
    o_ref[...] = _ln(x, nw_ref[...], nb_ref[...])


def _tc_final(edge_feat, scat, cnt0, cnt1, bpsi, norm_w, norm_b):
    BE = 4000
    grid = (E // BE,)
    row = lambda i: (i, 0)
    full = lambda i: (0, 0)
    return pl.pallas_call(
        _final_body,
        grid=grid,
        in_specs=[
            pl.BlockSpec((BE, EDGE_DIM), row),
            pl.BlockSpec((BE, EDGE_DIM), row),
            pl.BlockSpec((BE, 1), row),
            pl.BlockSpec((BE, 1), row),
            pl.BlockSpec((1, EDGE_DIM), full),
            pl.BlockSpec((1, EDGE_DIM), full),
            pl.BlockSpec((1, EDGE_DIM), full),
        ],
        out_specs=pl.BlockSpec((BE, EDGE_DIM), row),
        out_shape=jax.ShapeDtypeStruct((E, EDGE_DIM), jnp.float32),
    )(edge_feat, scat, cnt0[:, None], cnt1[:, None], bpsi[None],
      norm_w[None], norm_b[None])


def kernel(edge_feat, triplets, W1, b1, ln1_w, ln1_b, W2, b2, ln2_w, ln2_b,
           Wpsi, bpsi, norm_w, norm_b):
    e_ij = triplets[:, 0].astype(jnp.int32)
    e_kj = triplets[:, 1].astype(jnp.int32)
    t4 = triplets[:, 2:6]

    idx = jnp.concatenate([e_ij, e_kj], axis=0)
    rows = _sc_gather(edge_feat, idx)
    g1 = rows[:T]
    g2 = rows[T:]

    hw = _tc_mlp(g1, g2, t4, W1, b1, ln1_w, ln1_b, W2, b2, ln2_w, ln2_b, Wpsi)

    cnt2 = _sc_count(e_ij)
    scat = _sc_scatter(e_ij, hw)

    return _tc_final(edge_feat, scat, cnt2[0], cnt2[1], bpsi, norm_w, norm_b)


# speedup vs baseline: 4.0072x; 1.1353x over previous
import functools

import jax
import jax.numpy as jnp
from jax import lax
from jax.experimental import pallas as pl
from jax.experimental.pallas import tpu as pltpu
from jax.experimental.pallas import tpu_sc as plsc

E = 800000
T = 1600000
EDGE_DIM = 16
HIDDEN = 64


_NC, _NS, _L = 2, 16, 16
_NW = _NC * _NS


def _gather_body(table_hbm, ei_hbm, ek_hbm, g1_hbm, g2_hbm, idx_v, rows_v, sem):
    per_w = T // _NW
    blk = 2000
    n_blk = per_w // blk
    wid = lax.axis_index("s") * _NC + lax.axis_index("c")
    base = wid * per_w

    def body(i, carry):
        off = base + i * blk
        pltpu.sync_copy(ei_hbm.at[pl.ds(off, blk)], idx_v)
        pltpu.async_copy(table_hbm.at[idx_v], rows_v, sem).wait()
        pltpu.sync_copy(rows_v, g1_hbm.at[pl.ds(off, blk)])
        pltpu.sync_copy(ek_hbm.at[pl.ds(off, blk)], idx_v)
        pltpu.async_copy(table_hbm.at[idx_v], rows_v, sem).wait()
        pltpu.sync_copy(rows_v, g2_hbm.at[pl.ds(off, blk)])
        return carry

    lax.fori_loop(0, n_blk, body, 0)


def _sc_gather(table, eij, ekj):
    mesh = plsc.VectorSubcoreMesh(core_axis_name="c", subcore_axis_name="s")
    blk = 2000
    f = pl.kernel(
        _gather_body,
        out_type=(jax.ShapeDtypeStruct((T, EDGE_DIM), jnp.float32),
                  jax.ShapeDtypeStruct((T, EDGE_DIM), jnp.float32)),
        mesh=mesh,
        scratch_types=[
            pltpu.VMEM((blk,), jnp.int32),
            pltpu.VMEM((blk, EDGE_DIM), jnp.float32),
            pltpu.SemaphoreType.DMA,
        ],
        compiler_params=pltpu.CompilerParams(use_tc_tiling_on_sc=False),
    )
    return f(table, eij, ekj)


def _count_body(eij_hbm, out_hbm, idx_v, ones_v, zeros_v, cnt_sh, sem):
    del sem
    c = lax.axis_index("c")
    s = lax.axis_index("s")
    w = s * _NC + c

    def fill1(i, _):
        ones_v[pl.ds(i * 16, 16)] = jnp.full((16,), 1.0, jnp.float32)
        return 0

    def fill0(i, _):
        zeros_v[pl.ds(i * 16, 16)] = jnp.zeros((16,), jnp.float32)
        return 0

    lax.fori_loop(0, 125, fill1, 0)
    lax.fori_loop(0, 625, fill0, 0)
    for z in range(5):
        pltpu.sync_copy(zeros_v, cnt_sh.at[pl.ds(s * 50000 + z * 10000, 10000)])
    plsc.subcore_barrier()
    per_w = T // _NW
    for chunk in range(per_w // 2000):
        pltpu.sync_copy(eij_hbm.at[pl.ds(w * per_w + chunk * 2000, 2000)], idx_v)
        pltpu.sync_copy(ones_v, cnt_sh.at[idx_v], add=True)
    plsc.subcore_barrier()
    pltpu.sync_copy(cnt_sh.at[pl.ds(s * 50000, 50000)],
                    out_hbm.at[c, pl.ds(s * 50000, 50000)])


def _sc_count(eij):
    mesh = plsc.VectorSubcoreMesh(core_axis_name="c", subcore_axis_name="s")
    f = pl.kernel(
        _count_body,
        out_type=jax.ShapeDtypeStruct((2, E), jnp.float32),
        mesh=mesh,
        scratch_types=[
            pltpu.VMEM((2000,), jnp.int32),
            pltpu.VMEM((2000,), jnp.float32),
            pltpu.VMEM((10000,), jnp.float32),
            pltpu.VMEM_SHARED((E,), jnp.float32),
            pltpu.SemaphoreType.DMA,
        ],
        compiler_params=pltpu.CompilerParams(use_tc_tiling_on_sc=False),
    )
    return f(eij)


_CB = 100000
_NPASS = 4
_ACC = 102400
_DUMMY = 100000
_GRP = 5
_LF = 64
_LCAP = _LF + _GRP
_CAPB = 16 * _LCAP
_SIB = 2000


def _scatter_body(eij_hbm, hw_hbm, out_hbm, accum, idx_v, offs_v, ids_v,
                  rows_v, zeros_v, sem):
    c = lax.axis_index("c")
    s = lax.axis_index("s")
    iota = lax.iota(jnp.int32, 16)

    def zfill(i, _):
        zeros_v[i] = jnp.zeros((16,), jnp.float32)
        return 0

    lax.fori_loop(0, 400, zfill, 0)

    def _refill_one(i, _):
        offs_v[pl.ds(i * 16, 16)] = jnp.full((16,), _DUMMY, jnp.int32)
        ids_v[pl.ds(i * 16, 16)] = jnp.zeros((16,), jnp.int32)
        return 0

    def refill():
        lax.fori_loop(0, _CAPB // 16, _refill_one, 0)

    lane_base = iota * _LCAP

    def flush():
        pltpu.async_copy(hw_hbm.at[ids_v], rows_v, sem).wait()
        pltpu.sync_copy(rows_v, accum.at[offs_v], add=True)
        refill()

    for p in range(_NPASS):
        lo = (c * _NPASS + p) * _CB
        for z in range(16):
            pltpu.sync_copy(zeros_v, accum.at[pl.ds(s * 6400 + z * 400, 400)])
        refill()
        plsc.subcore_barrier()

        zero16 = jnp.zeros((16,), jnp.int32)
        one16 = jnp.full((16,), 1, jnp.int32)
        cb_v = jnp.full((16,), _CB, jnp.int32)
        lf_v = jnp.full((16,), _LF, jnp.int32)
        dummy_v = jnp.full((16,), _DUMMY, jnp.int32)
        lo_v = jnp.full((16,), lo, jnp.int32)

        def chunk_body(chunk, cnt_v):
            pltpu.sync_copy(
                eij_hbm.at[pl.ds(s * (T // _NS) + chunk * _SIB, _SIB)], idx_v)

            def group_body(g, cnt_v):
                for u in range(_GRP):
                    iv = idx_v[pl.ds((g * _GRP + u) * 16, 16)]
                    off = iv - lo_v
                    m = (off >= zero16) & (off < cb_v)
                    tid0 = s * (T // _NS) + chunk * _SIB + (g * _GRP + u) * 16
                    tids = jnp.full((16,), tid0, jnp.int32) + iota
                    pos = lane_base + cnt_v
                    plsc.store_scatter(offs_v, [pos], jnp.where(m, off, dummy_v))
                    plsc.store_scatter(ids_v, [pos], jnp.where(m, tids, zero16))
                    cnt_v = cnt_v + jnp.where(m, one16, zero16)

                anyfull = plsc.all_reduce_population_count(cnt_v >= lf_v)[0]

                def do_flush(c):
                    flush()
                    return zero16

                return lax.cond(anyfull > 0, do_flush, lambda c: c, cnt_v)

            return lax.fori_loop(0, _SIB // (16 * _GRP), group_body, cnt_v)

        cnt_v = lax.fori_loop(0, (T // _NS) // _SIB, chunk_body, zero16)
        del cnt_v
        flush()
        plsc.subcore_barrier()

        @pl.when(s < 15)
        def _():
            pltpu.sync_copy(accum.at[pl.ds(s * 6256, 6256)],
                            out_hbm.at[pl.ds(lo + s * 6256, 6256)])

        @pl.when(s == 15)
        def _():
            pltpu.sync_copy(accum.at[pl.ds(15 * 6256, _CB - 15 * 6256)],
                            out_hbm.at[pl.ds(lo + 15 * 6256, _CB - 15 * 6256)])

        plsc.subcore_barrier()


def _sc_scatter(eij, hw):
    mesh = plsc.VectorSubcoreMesh(core_axis_name="c", subcore_axis_name="s")
    f = pl.kernel(
        _scatter_body,
        out_type=jax.ShapeDtypeStruct((E, EDGE_DIM), jnp.float32),
        mesh=mesh,
        scratch_types=[
            pltpu.VMEM_SHARED((_ACC, EDGE_DIM), jnp.float32),
            pltpu.VMEM((_SIB,), jnp.int32),
            pltpu.VMEM((_CAPB,), jnp.int32),
            pltpu.VMEM((_CAPB,), jnp.int32),
            pltpu.VMEM((_CAPB, EDGE_DIM), jnp.float32),
            pltpu.VMEM((400, EDGE_DIM), jnp.float32),
            pltpu.SemaphoreType.DMA,
        ],
        compiler_params=pltpu.CompilerParams(use_tc_tiling_on_sc=False,
                                             needs_layout_passes=False),
    )
    return f(eij, hw)


def _silu(x):
    return x * jax.nn.sigmoid(x)


def _ln_mm(x, w, b, eps=1e-5):
    n = x.shape[-1]
    ones_n = jnp.full((n, n), 1.0 / n, jnp.float32)
    m = jnp.dot(x, ones_n, preferred_element_type=jnp.float32)
    d = x - m
    v = jnp.dot(d * d, ones_n, preferred_element_type=jnp.float32)
    return d * jax.lax.rsqrt(v + eps) * w + b


def _mlp_body(g1_ref, g2_ref, t4_ref, w1a_ref, w1b_ref, w1c_ref, b1_ref,
              ln1w_ref, ln1b_ref, w2_ref, b2_ref, ln2w_ref, ln2b_ref,
              wpsi_ref, o_ref):
    h = (jnp.dot(g1_ref[...], w1a_ref[...], preferred_element_type=jnp.float32)
         + jnp.dot(g2_ref[...], w1b_ref[...], preferred_element_type=jnp.float32)
         + jnp.dot(t4_ref[...], w1c_ref[...], preferred_element_type=jnp.float32)
         + b1_ref[...])
    h = _ln_mm(_silu(h), ln1w_ref[...], ln1b_ref[...])
    h = jnp.dot(h, w2_ref[...], preferred_element_type=jnp.float32) + b2_ref[...]
    h = _ln_mm(_silu(h), ln2w_ref[...], ln2b_ref[...])
    o_ref[...] = jnp.dot(h, wpsi_ref[...], preferred_element_type=jnp.float32)


def _tc_mlp(g1, g2, t4, W1, b1, ln1_w, ln1_b, W2, b2, ln2_w, ln2_b, Wpsi):
    BT = 4000
    grid = (T // BT,)
    w1a = W1[:EDGE_DIM]
    w1b = W1[EDGE_DIM:2 * EDGE_DIM]
    w1c = W1[2 * EDGE_DIM:]
    row = lambda i: (i, 0)
    full = lambda i: (0, 0)
    return pl.pallas_call(
        _mlp_body,
        grid=grid,
        in_specs=[
            pl.BlockSpec((BT, EDGE_DIM), row),
            pl.BlockSpec((BT, EDGE_DIM), row),
            pl.BlockSpec((BT, 4), row),
            pl.BlockSpec((EDGE_DIM, HIDDEN), full),
            pl.BlockSpec((EDGE_DIM, HIDDEN), full),
            pl.BlockSpec((4, HIDDEN), full),
            pl.BlockSpec((1, HIDDEN), full),
            pl.BlockSpec((1, HIDDEN), full),
            pl.BlockSpec((1, HIDDEN), full),
            pl.BlockSpec((HIDDEN, HIDDEN), full),
            pl.BlockSpec((1, HIDDEN), full),
            pl.BlockSpec((1, HIDDEN), full),
            pl.BlockSpec((1, HIDDEN), full),
            pl.BlockSpec((HIDDEN, EDGE_DIM), full),
        ],
        out_specs=pl.BlockSpec((BT, EDGE_DIM), row),
        out_shape=jax.ShapeDtypeStruct((T, EDGE_DIM), jnp.float32),
    )(g1, g2, t4, w1a, w1b, w1c, b1[None], ln1_w[None], ln1_b[None],
      W2, b2[None], ln2_w[None], ln2_b[None], Wpsi)


def _final_body(ef_ref, sc_ref, c0_ref, c1_ref, bpsi_ref, nw_ref, nb_ref, o_ref):
    cnt = jnp.maximum(c0_ref[...] + c1_ref[...], 1.0)
    x = ef_ref[...] + sc_ref[...] / cnt + bpsi_ref[...]
    o_ref[...] = _ln_mm(x, nw_ref[...], nb_ref[...])


def _tc_final(edge_feat, scat, cnt0, cnt1, bpsi, norm_w, norm_b):
    BE = 4000
    grid = (E // BE,)
    row = lambda i: (i, 0)
    full = lambda i: (0, 0)
    return pl.pallas_call(
        _final_body,
        grid=grid,
        in_specs=[
            pl.BlockSpec((BE, EDGE_DIM), row),
            pl.BlockSpec((BE, EDGE_DIM), row),
            pl.BlockSpec((BE, 1), row),
            pl.BlockSpec((BE, 1), row),
            pl.BlockSpec((1, EDGE_DIM), full),
            pl.BlockSpec((1, EDGE_DIM), full),
            pl.BlockSpec((1, EDGE_DIM), full),
        ],
        out_specs=pl.BlockSpec((BE, EDGE_DIM), row),
        out_shape=jax.ShapeDtypeStruct((E, EDGE_DIM), jnp.float32),
    )(edge_feat, scat, cnt0[:, None], cnt1[:, None], bpsi[None],
      norm_w[None], norm_b[None])


def kernel(edge_feat, triplets, W1, b1, ln1_w, ln1_b, W2, b2, ln2_w, ln2_b,
           Wpsi, bpsi, norm_w, norm_b):
    e_ij = triplets[:, 0].astype(jnp.int32)
    e_kj = triplets[:, 1].astype(jnp.int32)
    t4 = triplets[:, 2:6]

    g1, g2 = _sc_gather(edge_feat, e_ij, e_kj)

    hw = _tc_mlp(g1, g2, t4, W1, b1, ln1_w, ln1_b, W2, b2, ln2_w, ln2_b, Wpsi)

    cnt2 = _sc_count(e_ij)
    scat = _sc_scatter(e_ij, hw)

    return _tc_final(edge_feat, scat, cnt2[0], cnt2[1], bpsi, norm_w, norm_b)
